# Initial kernel scaffold; baseline (speedup 1.0000x reference)
import jax
import jax.numpy as jnp
from jax.experimental import pallas as pl


def kernel(features_1, edge_index_1, edge_attr_1, batch_1, features_2, edge_index_2, edge_attr_2, batch_2, W0, b0, W1, W2, W3, Wc1, Wc2, Wc3, p1, p2, p3, Wr0, Wr1, Wr2, Wr3, Wm1, bm1, Wm2, bm2):
    raise NotImplementedError("write your pallas kernel here")


# speedup vs baseline: 1.2649x; 1.2649x over previous
import functools

import jax
import jax.numpy as jnp
from jax import lax
from jax.experimental import pallas as pl
from jax.experimental.pallas import tpu as pltpu
from jax.experimental.pallas import tpu_sc as plsc

EPS = 1e-9
NC, NS, L = 2, 16, 16
NW = NC * NS
CHUNK = 512
SCHUNK = 2048
NHID = 64


def _dyn_splat(v, i):
    idx = jnp.full((L,), i, jnp.int32)
    return lax.gather(
        v, idx[:, None],
        lax.GatherDimensionNumbers(offset_dims=(), collapsed_slice_dims=(0,),
                                   start_index_map=(0,)),
        (1,), mode=lax.GatherScatterMode.PROMISE_IN_BOUNDS)


def _chunk_ranges(total, maxc):
    offs, off = [], 0
    while off < total:
        size = maxc
        while size > total - off:
            size //= 2
        offs.append((off, size))
        off += size
    return offs


@functools.lru_cache(maxsize=None)
def _make_feature_kernel(S_TOT, T_TOT, TOT):
    nch = TOT // (NW * CHUNK)
    rps = T_TOT // NS
    mesh = plsc.VectorSubcoreMesh(core_axis_name="c", subcore_axis_name="s")

    tps = S_TOT // NS
    tpt = T_TOT // NS

    @functools.partial(
        pl.kernel,
        out_type=jax.ShapeDtypeStruct((NC * T_TOT, NHID), jnp.float32),
        mesh=mesh,
        compiler_params=pltpu.CompilerParams(needs_layout_passes=False, use_tc_tiling_on_sc=False),
        scratch_types=[
            pltpu.VMEM((CHUNK,), jnp.int32),
            pltpu.VMEM((CHUNK,), jnp.int32),
            pltpu.VMEM((CHUNK,), jnp.float32),
            pltpu.VMEM((CHUNK,), jnp.float32),
            pltpu.VMEM((CHUNK,), jnp.float32),
            pltpu.VMEM((CHUNK, NHID), jnp.float32),
            pltpu.VMEM((max(tps, tpt),), jnp.float32),
            pltpu.VMEM_SHARED((S_TOT,), jnp.float32),
            pltpu.VMEM_SHARED((T_TOT,), jnp.float32),
            pltpu.VMEM_SHARED((T_TOT, NHID), jnp.float32),
            pltpu.SemaphoreType.DMA,
        ],
    )
    def kern(x_hbm, g_hbm, s_hbm, bw_hbm, gs_hbm, ss_hbm, out_hbm,
             g_buf, s_buf, w_buf, gs_v, ss_v, rows, bounce,
             gs_sh, ss_sh, acc, sem):
        cid = lax.axis_index("c")
        sid = lax.axis_index("s")
        wid = sid * NC + cid

        pltpu.sync_copy(gs_hbm.at[pl.ds(sid * tps, tps)],
                        bounce.at[pl.ds(0, tps)])
        pltpu.sync_copy(bounce.at[pl.ds(0, tps)],
                        gs_sh.at[pl.ds(sid * tps, tps)])
        pltpu.sync_copy(ss_hbm.at[pl.ds(sid * tpt, tpt)],
                        bounce.at[pl.ds(0, tpt)])
        pltpu.sync_copy(bounce.at[pl.ds(0, tpt)],
                        ss_sh.at[pl.ds(sid * tpt, tpt)])

        def zero_body(k, c):
            for j in range(NHID // L):
                rows[k, pl.ds(j * L, L)] = jnp.zeros((L,), jnp.float32)
            return c

        lax.fori_loop(0, CHUNK, zero_body, 0)
        for off, size in _chunk_ranges(rps, CHUNK):
            pltpu.sync_copy(rows.at[pl.ds(0, size)],
                            acc.at[pl.ds(sid * rps + off, size)])
        plsc.subcore_barrier()

        def chunk_body(it, c):
            base = (it * NW + wid) * CHUNK
            pltpu.sync_copy(g_hbm.at[pl.ds(base, CHUNK)], g_buf)
            pltpu.sync_copy(s_hbm.at[pl.ds(base, CHUNK)], s_buf)
            pltpu.sync_copy(bw_hbm.at[pl.ds(base, CHUNK)], w_buf)
            cp = pltpu.async_copy(x_hbm.at[g_buf], rows, sem)
            pltpu.sync_copy(gs_sh.at[g_buf], gs_v)
            pltpu.sync_copy(ss_sh.at[s_buf], ss_v)
            cp.wait()

            def scale_body(k, c2):
                b16 = k * L
                wv = (w_buf[pl.ds(b16, L)]
                      * gs_v[pl.ds(b16, L)]
                      * ss_v[pl.ds(b16, L)])
                for i in range(L):
                    spl = _dyn_splat(wv, i)
                    for j in range(NHID // L):
                        sl = pl.ds(j * L, L)
                        rows[b16 + i, sl] = rows[b16 + i, sl] * spl
                return c2

            lax.fori_loop(0, CHUNK // L, scale_body, 0)
            pltpu.sync_copy(rows, acc.at[s_buf], add=True)
            return c

        lax.fori_loop(0, nch, chunk_body, 0)
        plsc.subcore_barrier()
        for off, size in _chunk_ranges(rps, CHUNK):
            pltpu.sync_copy(acc.at[pl.ds(sid * rps + off, size)],
                            rows.at[pl.ds(0, size)])
            pltpu.sync_copy(
                rows.at[pl.ds(0, size)],
                out_hbm.at[pl.ds(cid * T_TOT + sid * rps + off, size)])

    return kern


@functools.lru_cache(maxsize=None)
def _make_scalar_kernel(T_TOT, TOT):
    nch = TOT // (NW * SCHUNK)
    eps = T_TOT // NS
    mesh = plsc.VectorSubcoreMesh(core_axis_name="c", subcore_axis_name="s")

    @functools.partial(
        pl.kernel,
        out_type=jax.ShapeDtypeStruct((NC * T_TOT,), jnp.float32),
        mesh=mesh,
        compiler_params=pltpu.CompilerParams(needs_layout_passes=False, use_tc_tiling_on_sc=False),
        scratch_types=[
            pltpu.VMEM((SCHUNK,), jnp.int32),
            pltpu.VMEM((SCHUNK,), jnp.float32),
            pltpu.VMEM((T_TOT // NS,), jnp.float32),
            pltpu.VMEM_SHARED((T_TOT,), jnp.float32),
        ],
    )
    def kern(i_hbm, w_hbm, out_hbm, i_buf, w_buf, bounce, acc):
        cid = lax.axis_index("c")
        sid = lax.axis_index("s")
        wid = sid * NC + cid

        def zero_body(k, c):
            bounce[pl.ds(k * L, L)] = jnp.zeros((L,), jnp.float32)
            return c

        lax.fori_loop(0, eps // L, zero_body, 0)
        pltpu.sync_copy(bounce, acc.at[pl.ds(sid * eps, eps)])
        plsc.subcore_barrier()

        def body(it, c):
            base = (it * NW + wid) * SCHUNK
            pltpu.sync_copy(i_hbm.at[pl.ds(base, SCHUNK)], i_buf)
            pltpu.sync_copy(w_hbm.at[pl.ds(base, SCHUNK)], w_buf)
            pltpu.sync_copy(w_buf, acc.at[i_buf], add=True)
            return c

        lax.fori_loop(0, nch, body, 0)
        plsc.subcore_barrier()
        pltpu.sync_copy(acc.at[pl.ds(sid * eps, eps)], bounce)
        pltpu.sync_copy(bounce,
                        out_hbm.at[pl.ds(cid * T_TOT + sid * eps, eps)])

    return kern


def _pad_pairs(g, s, bw, S_TOT, T_TOT, TOT):
    M = g.shape[0]
    npad = TOT - M
    if npad:
        ar = jnp.arange(npad, dtype=jnp.int32)
        g = jnp.concatenate([g, ar % S_TOT])
        s = jnp.concatenate([s, ar % T_TOT])
        bw = jnp.concatenate([bw, jnp.zeros((npad,), bw.dtype)])
    return g, s, bw, TOT


def _seg_pass(x, g, s, bw, gs, ss, S_TOT, T_TOT, TOT):
    g, s, bw, TOT = _pad_pairs(g, s, bw, S_TOT, T_TOT, TOT)
    kern = _make_feature_kernel(S_TOT, T_TOT, TOT)
    parts = kern(x, g, s, bw, gs, ss)
    return parts[:T_TOT] + parts[T_TOT:]


def _seg_scalar(i, w, T_TOT, TOT):
    i, _, w, TOT = _pad_pairs(i, i, w, T_TOT, T_TOT, TOT)
    kern = _make_scalar_kernel(T_TOT, TOT)
    parts = kern(i, w)
    return parts[:T_TOT] + parts[T_TOT:]


def _leaky(x):
    return jnp.where(x > 0, x, 0.2 * x)


def _readout(x, Wr):
    m = jnp.mean(x, axis=0, keepdims=True)
    gate = jax.nn.sigmoid(x @ Wr @ m.T)
    return jnp.sum(gate * x, axis=0, keepdims=True)


def _cross(x1, x2, W):
    a12 = jax.nn.softmax((x1 @ W) @ x2.T, axis=1)
    a21 = jax.nn.softmax((x2 @ W) @ x1.T, axis=1)
    return a12 @ x2, a21 @ x1


def _pool(ef, k, p):
    score = jnp.tanh(ef @ p / (jnp.linalg.norm(p) + EPS))
    vals, idx = lax.top_k(score, k)
    pooled = ef[idx] * vals[:, None]
    num = ef.shape[0]
    mapping = jnp.zeros((num,), jnp.int32).at[idx].set(
        jnp.arange(k, dtype=jnp.int32))
    keep = jnp.zeros((num,), ef.dtype).at[idx].set(1.0)
    return pooled, mapping, keep


def _xtab(x1, x2, P):
    o1 = jnp.zeros((P, x1.shape[1]), x1.dtype).at[:x1.shape[0]].set(x1)
    o2 = jnp.zeros((P, x2.shape[1]), x2.dtype).at[:x2.shape[0]].set(x2)
    return jnp.concatenate([o1, o2], axis=0)


def kernel(features_1, edge_index_1, edge_attr_1, batch_1, features_2,
           edge_index_2, edge_attr_2, batch_2, W0, b0, W1, W2, W3, Wc1, Wc2,
           Wc3, p1, p2, p3, Wr0, Wr1, Wr2, Wr3, Wm1, bm1, Wm2, bm2):
    n = features_1.shape[0]
    K1 = int(0.2 * n); K2 = K1 // 2; K3 = K2 // 2
    NP = -(-n // 1024) * 1024
    E_ = edge_index_1.shape[1]
    M_ = E_ + n
    FTOT = -(-(2 * M_) // (NW * CHUNK)) * (NW * CHUNK)
    STOT = -(-(4 * M_) // (NW * SCHUNK)) * (NW * SCHUNK)
    src1, dst1 = edge_index_1[0], edge_index_1[1]
    src2, dst2 = edge_index_2[0], edge_index_2[1]
    ew1, ew2 = edge_attr_1, edge_attr_2

    ones2NP = jnp.ones((2 * NP,), jnp.float32)

    idx4 = jnp.concatenate([src1, dst1 + NP, src2 + 2 * NP, dst2 + 3 * NP])
    w4 = jnp.concatenate([ew1, ew1, ew2, ew2])
    deg = _seg_scalar(idx4, w4, 4 * NP, STOT)
    rs = lax.rsqrt(deg + EPS)
    rs_s1, rs_d1 = rs[0:NP], rs[NP:2 * NP]
    rs_s2, rs_d2 = rs[2 * NP:3 * NP], rs[3 * NP:4 * NP]

    h1 = features_1 @ W0
    h2 = features_2 @ W0
    g = jnp.concatenate([src1, src2 + NP])
    s = jnp.concatenate([dst1, dst2 + NP])
    bw = jnp.concatenate([ew1, ew2])
    out = _seg_pass(_xtab(h1, h2, NP), g, s, bw,
                    jnp.concatenate([rs_s1, rs_s2]),
                    jnp.concatenate([rs_d1, rs_d2]), 2 * NP, 2 * NP, FTOT)
    f1 = _leaky(out[:n] + b0)
    f2 = _leaky(out[NP:NP + n] + b0)
    s0 = jnp.concatenate([_readout(f1, Wr0), _readout(f2, Wr0)], axis=1)

    ar_n = jnp.arange(n, dtype=jnp.int32)
    n1 = jnp.concatenate([src1, ar_n]); h1i = jnp.concatenate([dst1, ar_n])
    a1 = jnp.concatenate([ew1, jnp.ones((n,), jnp.float32)])
    n2 = jnp.concatenate([src2, ar_n]); h2i = jnp.concatenate([dst2, ar_n])
    a2 = jnp.concatenate([ew2, jnp.ones((n,), jnp.float32)])

    idx4 = jnp.concatenate([h1i, n1 + NP, h2i + 2 * NP, n2 + 3 * NP])
    w4 = jnp.concatenate([a1, a1, a2, a2])
    deg = _seg_scalar(idx4, w4, 4 * NP, STOT)
    inv = 1.0 / (deg + EPS)
    invB1, invD1 = inv[0:NP], inv[NP:2 * NP]
    invB2, invD2 = inv[2 * NP:3 * NP], inv[3 * NP:4 * NP]

    hh1 = f1 @ W1; hh2 = f2 @ W1
    g = jnp.concatenate([n1, n2 + NP])
    s = jnp.concatenate([h1i, h2i + NP])
    bw = jnp.concatenate([a1, a2])
    ef_int = _seg_pass(_xtab(hh1, hh2, NP), g, s, bw, ones2NP,
                       jnp.concatenate([invB1, invB2]), 2 * NP, 2 * NP, FTOT)
    gr = jnp.concatenate([h1i, h2i + NP])
    sr = jnp.concatenate([n1, n2 + NP])
    out = _seg_pass(ef_int, gr, sr, bw, ones2NP,
                    jnp.concatenate([invD1, invD2]), 2 * NP, 2 * NP, FTOT)
    f1c = _leaky(out[:n]); f2c = _leaky(out[NP:NP + n])

    ef = _seg_pass(_xtab(f1c, f2c, NP), g, s, bw, ones2NP,
                   jnp.concatenate([invB1, invB2]), 2 * NP, 2 * NP, FTOT)
    ef1 = ef[:n]; ef2 = ef[NP:NP + n]

    e1, map1, keep1 = _pool(ef1, K1, p1)
    e2, map2, keep2 = _pool(ef2, K1, p1)
    h1p = map1[h1i]; a1p = a1 * keep1[h1i]
    h2p = map2[h2i]; a2p = a2 * keep2[h2i]
    x1, x2 = _cross(e1, e2, Wc1)
    s1 = jnp.concatenate([_readout(x1, Wr1), _readout(x2, Wr1)], axis=1)

    def he_layer(x1, x2, h1p, a1p, h2p, a2p, K, W):
        idx4 = jnp.concatenate([n1, h1p + NP, n2 + 2 * NP, h2p + 3 * NP])
        w4 = jnp.concatenate([a1p, a1p, a2p, a2p])
        deg = _seg_scalar(idx4, w4, 4 * NP, STOT)
        inv = 1.0 / (deg + EPS)
        iD1, iB1 = inv[0:NP], inv[NP:2 * NP]
        iD2, iB2 = inv[2 * NP:3 * NP], inv[3 * NP:4 * NP]
        g = jnp.concatenate([h1p, h2p + NP])
        s = jnp.concatenate([n1, n2 + NP])
        bw = jnp.concatenate([a1p, a2p])
        nf = _seg_pass(_xtab(x1, x2, NP), g, s, bw, ones2NP,
                       jnp.concatenate([iD1, iD2]), 2 * NP, 2 * NP, FTOT)
        out = _seg_pass(nf, s, g, bw, ones2NP,
                        jnp.concatenate([iB1, iB2]), 2 * NP, 2 * NP, FTOT)
        o1 = _leaky(out[:K] @ W)
        o2 = _leaky(out[NP:NP + K] @ W)
        return o1, o2

    g1o, g2o = he_layer(x1, x2, h1p, a1p, h2p, a2p, K1, W2)
    e1, m1b, k1b = _pool(g1o, K2, p2)
    e2, m2b, k2b = _pool(g2o, K2, p2)
    h1p2 = m1b[h1p]; a1p2 = a1p * k1b[h1p]
    h2p2 = m2b[h2p]; a2p2 = a2p * k2b[h2p]
    x1, x2 = _cross(e1, e2, Wc2)
    s2 = jnp.concatenate([_readout(x1, Wr2), _readout(x2, Wr2)], axis=1)

    g1o, g2o = he_layer(x1, x2, h1p2, a1p2, h2p2, a2p2, K2, W3)
    e1, _, _ = _pool(g1o, K3, p3)
    e2, _, _ = _pool(g2o, K3, p3)
    x1, x2 = _cross(e1, e2, Wc3)
    s3 = jnp.concatenate([_readout(x1, Wr3), _readout(x2, Wr3)], axis=1)

    scores = jnp.concatenate([s0, s1, s2, s3], axis=1)
    hmid = _leaky(scores @ Wm1 + bm1)
    return hmid @ Wm2 + bm2


# speedup vs baseline: 2.6427x; 2.0893x over previous
import functools

import jax
import jax.numpy as jnp
from jax import lax
from jax.experimental import pallas as pl
from jax.experimental.pallas import tpu as pltpu
from jax.experimental.pallas import tpu_sc as plsc

EPS = 1e-9
NC, NS, L = 2, 16, 16
CHUNK = 512
SCHUNK = 2048
NHID = 64


def _dyn_splat(v, i):
    idx = jnp.full((L,), i, jnp.int32)
    return lax.gather(
        v, idx[:, None],
        lax.GatherDimensionNumbers(offset_dims=(), collapsed_slice_dims=(0,),
                                   start_index_map=(0,)),
        (1,), mode=lax.GatherScatterMode.PROMISE_IN_BOUNDS)


def _chunk_ranges(total, maxc):
    offs, off = [], 0
    while off < total:
        size = maxc
        while size > total - off:
            size //= 2
        offs.append((off, size))
        off += size
    return offs


@functools.lru_cache(maxsize=None)
def _make_feature_kernel(SP, TP, TOTP):
    nch = TOTP // (NS * CHUNK)
    rps = TP // NS
    spr = SP // NS
    tps = SP // NS
    tpt = TP // NS
    mesh = plsc.VectorSubcoreMesh(core_axis_name="c", subcore_axis_name="s")

    @functools.partial(
        pl.kernel,
        out_type=jax.ShapeDtypeStruct((NC * TP, NHID), jnp.float32),
        mesh=mesh,
        compiler_params=pltpu.CompilerParams(
            needs_layout_passes=False, use_tc_tiling_on_sc=False),
        scratch_types=[
            pltpu.VMEM((CHUNK,), jnp.int32),
            pltpu.VMEM((CHUNK,), jnp.int32),
            pltpu.VMEM((CHUNK,), jnp.float32),
            pltpu.VMEM((CHUNK,), jnp.float32),
            pltpu.VMEM((CHUNK,), jnp.float32),
            pltpu.VMEM((CHUNK, NHID), jnp.float32),
            pltpu.VMEM((max(tps, tpt),), jnp.float32),
            pltpu.VMEM_SHARED((SP, NHID), jnp.float32),
            pltpu.VMEM_SHARED((SP,), jnp.float32),
            pltpu.VMEM_SHARED((TP,), jnp.float32),
            pltpu.VMEM_SHARED((TP, NHID), jnp.float32),
            pltpu.SemaphoreType.DMA,
        ],
    )
    def kern(x_hbm, g_hbm, s_hbm, bw_hbm, gs_hbm, ss_hbm, out_hbm,
             g_buf, s_buf, w_buf, gs_v, ss_v, rows, bounce,
             x_sh, gs_sh, ss_sh, acc, sem):
        cid = lax.axis_index("c")
        sid = lax.axis_index("s")

        for off, size in _chunk_ranges(spr, CHUNK):
            pltpu.sync_copy(x_hbm.at[pl.ds(cid * SP + sid * spr + off, size)],
                            rows.at[pl.ds(0, size)])
            pltpu.sync_copy(rows.at[pl.ds(0, size)],
                            x_sh.at[pl.ds(sid * spr + off, size)])
        pltpu.sync_copy(gs_hbm.at[pl.ds(cid * SP + sid * tps, tps)],
                        bounce.at[pl.ds(0, tps)])
        pltpu.sync_copy(bounce.at[pl.ds(0, tps)],
                        gs_sh.at[pl.ds(sid * tps, tps)])
        pltpu.sync_copy(ss_hbm.at[pl.ds(cid * TP + sid * tpt, tpt)],
                        bounce.at[pl.ds(0, tpt)])
        pltpu.sync_copy(bounce.at[pl.ds(0, tpt)],
                        ss_sh.at[pl.ds(sid * tpt, tpt)])

        def zero_body(k, c):
            for j in range(NHID // L):
                rows[k, pl.ds(j * L, L)] = jnp.zeros((L,), jnp.float32)
            return c

        lax.fori_loop(0, CHUNK, zero_body, 0)
        for off, size in _chunk_ranges(rps, CHUNK):
            pltpu.sync_copy(rows.at[pl.ds(0, size)],
                            acc.at[pl.ds(sid * rps + off, size)])
        plsc.subcore_barrier()

        def chunk_body(it, c):
            base = cid * TOTP + (it * NS + sid) * CHUNK
            pltpu.sync_copy(g_hbm.at[pl.ds(base, CHUNK)], g_buf)
            pltpu.sync_copy(s_hbm.at[pl.ds(base, CHUNK)], s_buf)
            pltpu.sync_copy(bw_hbm.at[pl.ds(base, CHUNK)], w_buf)
            cp = pltpu.async_copy(x_sh.at[g_buf], rows, sem)
            pltpu.sync_copy(gs_sh.at[g_buf], gs_v)
            pltpu.sync_copy(ss_sh.at[s_buf], ss_v)
            cp.wait()

            def scale_body(k, c2):
                b16 = k * L
                wv = (w_buf[pl.ds(b16, L)]
                      * gs_v[pl.ds(b16, L)]
                      * ss_v[pl.ds(b16, L)])
                for i in range(L):
                    spl = _dyn_splat(wv, i)
                    for j in range(NHID // L):
                        sl = pl.ds(j * L, L)
                        rows[b16 + i, sl] = rows[b16 + i, sl] * spl
                return c2

            lax.fori_loop(0, CHUNK // L, scale_body, 0)
            pltpu.sync_copy(rows, acc.at[s_buf], add=True)
            return c

        lax.fori_loop(0, nch, chunk_body, 0)
        plsc.subcore_barrier()
        for off, size in _chunk_ranges(rps, CHUNK):
            pltpu.sync_copy(acc.at[pl.ds(sid * rps + off, size)],
                            rows.at[pl.ds(0, size)])
            pltpu.sync_copy(
                rows.at[pl.ds(0, size)],
                out_hbm.at[pl.ds(cid * TP + sid * rps + off, size)])

    return kern


@functools.lru_cache(maxsize=None)
def _make_scalar_kernel(TP2, TOT2):
    nch = TOT2 // (NS * SCHUNK)
    eps = TP2 // NS
    mesh = plsc.VectorSubcoreMesh(core_axis_name="c", subcore_axis_name="s")

    @functools.partial(
        pl.kernel,
        out_type=jax.ShapeDtypeStruct((NC * TP2,), jnp.float32),
        mesh=mesh,
        compiler_params=pltpu.CompilerParams(
            needs_layout_passes=False, use_tc_tiling_on_sc=False),
        scratch_types=[
            pltpu.VMEM((SCHUNK,), jnp.int32),
            pltpu.VMEM((SCHUNK,), jnp.float32),
            pltpu.VMEM((TP2 // NS,), jnp.float32),
            pltpu.VMEM_SHARED((TP2,), jnp.float32),
        ],
    )
    def kern(i_hbm, w_hbm, out_hbm, i_buf, w_buf, bounce, acc):
        cid = lax.axis_index("c")
        sid = lax.axis_index("s")

        def zero_body(k, c):
            bounce[pl.ds(k * L, L)] = jnp.zeros((L,), jnp.float32)
            return c

        lax.fori_loop(0, eps // L, zero_body, 0)
        pltpu.sync_copy(bounce, acc.at[pl.ds(sid * eps, eps)])
        plsc.subcore_barrier()

        def body(it, c):
            base = cid * TOT2 + (it * NS + sid) * SCHUNK
            pltpu.sync_copy(i_hbm.at[pl.ds(base, SCHUNK)], i_buf)
            pltpu.sync_copy(w_hbm.at[pl.ds(base, SCHUNK)], w_buf)
            pltpu.sync_copy(w_buf, acc.at[i_buf], add=True)
            return c

        lax.fori_loop(0, nch, body, 0)
        plsc.subcore_barrier()
        pltpu.sync_copy(acc.at[pl.ds(sid * eps, eps)], bounce)
        pltpu.sync_copy(bounce,
                        out_hbm.at[pl.ds(cid * TP2 + sid * eps, eps)])

    return kern


def _pad1(g, s, bw, SP, TP, TOT):
    npad = TOT - g.shape[0]
    if npad:
        ar = jnp.arange(npad, dtype=jnp.int32)
        g = jnp.concatenate([g, ar % SP])
        s = jnp.concatenate([s, ar % TP])
        bw = jnp.concatenate([bw, jnp.zeros((npad,), bw.dtype)])
    return g, s, bw


def _seg_pass(x1, x2, pairs1, pairs2, gs1, gs2, ss1, ss2, SP, TP, TOTP):
    g1, s1, bw1 = _pad1(*pairs1, SP, TP, TOTP)
    g2, s2, bw2 = _pad1(*pairs2, SP, TP, TOTP)
    g = jnp.concatenate([g1, g2])
    s = jnp.concatenate([s1, s2])
    bw = jnp.concatenate([bw1, bw2])
    x = _xtab(x1, x2, SP)
    gs = jnp.concatenate([gs1[:SP], gs2[:SP]])
    ss = jnp.concatenate([ss1[:TP], ss2[:TP]])
    kern = _make_feature_kernel(SP, TP, TOTP)
    return kern(x, g, s, bw, gs, ss)


def _seg_scalar(i1, w1, i2, w2, TP2, TOT2):
    i1, _, w1 = _pad1(i1, i1, w1, TP2, TP2, TOT2)
    i2, _, w2 = _pad1(i2, i2, w2, TP2, TP2, TOT2)
    i = jnp.concatenate([i1, i2])
    w = jnp.concatenate([w1, w2])
    kern = _make_scalar_kernel(TP2, TOT2)
    return kern(i, w)


def _leaky(x):
    return jnp.where(x > 0, x, 0.2 * x)


def _readout(x, Wr):
    m = jnp.mean(x, axis=0, keepdims=True)
    gate = jax.nn.sigmoid(x @ Wr @ m.T)
    return jnp.sum(gate * x, axis=0, keepdims=True)


def _cross(x1, x2, W):
    a12 = jax.nn.softmax((x1 @ W) @ x2.T, axis=1)
    a21 = jax.nn.softmax((x2 @ W) @ x1.T, axis=1)
    return a12 @ x2, a21 @ x1


def _pool(ef, k, p):
    score = jnp.tanh(ef @ p / (jnp.linalg.norm(p) + EPS))
    vals, idx = lax.top_k(score, k)
    pooled = ef[idx] * vals[:, None]
    num = ef.shape[0]
    mapping = jnp.zeros((num,), jnp.int32).at[idx].set(
        jnp.arange(k, dtype=jnp.int32))
    keep = jnp.zeros((num,), ef.dtype).at[idx].set(1.0)
    return pooled, mapping, keep


def _xtab(x1, x2, P):
    o1 = jnp.zeros((P, x1.shape[1]), x1.dtype).at[:x1.shape[0]].set(x1)
    o2 = jnp.zeros((P, x2.shape[1]), x2.dtype).at[:x2.shape[0]].set(x2)
    return jnp.concatenate([o1, o2], axis=0)


def kernel(features_1, edge_index_1, edge_attr_1, batch_1, features_2,
           edge_index_2, edge_attr_2, batch_2, W0, b0, W1, W2, W3, Wc1, Wc2,
           Wc3, p1, p2, p3, Wr0, Wr1, Wr2, Wr3, Wm1, bm1, Wm2, bm2):
    n = features_1.shape[0]
    K1 = int(0.2 * n); K2 = K1 // 2; K3 = K2 // 2
    NP = -(-n // 1024) * 1024
    KP = -(-K1 // 1024) * 1024
    E_ = edge_index_1.shape[1]
    M_ = E_ + n
    FTOT = -(-M_ // (NS * CHUNK)) * (NS * CHUNK)
    STOT = -(-(2 * M_) // (NS * SCHUNK)) * (NS * SCHUNK)
    src1, dst1 = edge_index_1[0], edge_index_1[1]
    src2, dst2 = edge_index_2[0], edge_index_2[1]
    ew1, ew2 = edge_attr_1, edge_attr_2

    onesNP = jnp.ones((NP,), jnp.float32)
    onesKP = jnp.ones((KP,), jnp.float32)

    deg = _seg_scalar(jnp.concatenate([src1, dst1 + NP]),
                      jnp.concatenate([ew1, ew1]),
                      jnp.concatenate([src2, dst2 + NP]),
                      jnp.concatenate([ew2, ew2]), 2 * NP, STOT)
    rs = lax.rsqrt(deg + EPS)
    rs_s1, rs_d1 = rs[0:NP], rs[NP:2 * NP]
    rs_s2, rs_d2 = rs[2 * NP:3 * NP], rs[3 * NP:4 * NP]

    h1 = features_1 @ W0
    h2 = features_2 @ W0
    out = _seg_pass(h1, h2, (src1, dst1, ew1), (src2, dst2, ew2),
                    rs_s1, rs_s2, rs_d1, rs_d2, NP, NP, FTOT)
    f1 = _leaky(out[:n] + b0)
    f2 = _leaky(out[NP:NP + n] + b0)
    s0 = jnp.concatenate([_readout(f1, Wr0), _readout(f2, Wr0)], axis=1)

    ar_n = jnp.arange(n, dtype=jnp.int32)
    n1 = jnp.concatenate([src1, ar_n]); h1i = jnp.concatenate([dst1, ar_n])
    a1 = jnp.concatenate([ew1, jnp.ones((n,), jnp.float32)])
    n2 = jnp.concatenate([src2, ar_n]); h2i = jnp.concatenate([dst2, ar_n])
    a2 = jnp.concatenate([ew2, jnp.ones((n,), jnp.float32)])

    deg = _seg_scalar(jnp.concatenate([h1i, n1 + NP]),
                      jnp.concatenate([a1, a1]),
                      jnp.concatenate([h2i, n2 + NP]),
                      jnp.concatenate([a2, a2]), 2 * NP, STOT)
    inv = 1.0 / (deg + EPS)
    invB1, invD1 = inv[0:NP], inv[NP:2 * NP]
    invB2, invD2 = inv[2 * NP:3 * NP], inv[3 * NP:4 * NP]

    hh1 = f1 @ W1; hh2 = f2 @ W1
    ef_int = _seg_pass(hh1, hh2, (n1, h1i, a1), (n2, h2i, a2),
                       onesNP, onesNP, invB1, invB2, NP, NP, FTOT)
    out = _seg_pass(ef_int[:NP], ef_int[NP:], (h1i, n1, a1), (h2i, n2, a2),
                    onesNP, onesNP, invD1, invD2, NP, NP, FTOT)
    f1c = _leaky(out[:n]); f2c = _leaky(out[NP:NP + n])

    ef = _seg_pass(f1c, f2c, (n1, h1i, a1), (n2, h2i, a2),
                   onesNP, onesNP, invB1, invB2, NP, NP, FTOT)
    ef1 = ef[:n]; ef2 = ef[NP:NP + n]

    e1, map1, keep1 = _pool(ef1, K1, p1)
    e2, map2, keep2 = _pool(ef2, K1, p1)
    h1p = map1[h1i]; a1p = a1 * keep1[h1i]
    h2p = map2[h2i]; a2p = a2 * keep2[h2i]
    x1, x2 = _cross(e1, e2, Wc1)
    s1 = jnp.concatenate([_readout(x1, Wr1), _readout(x2, Wr1)], axis=1)

    def he_layer(x1, x2, h1p, a1p, h2p, a2p, K, W):
        deg = _seg_scalar(jnp.concatenate([n1, h1p + NP]),
                          jnp.concatenate([a1p, a1p]),
                          jnp.concatenate([n2, h2p + NP]),
                          jnp.concatenate([a2p, a2p]), 2 * NP, STOT)
        inv = 1.0 / (deg + EPS)
        iD1, iB1 = inv[0:NP], inv[NP:2 * NP]
        iD2, iB2 = inv[2 * NP:3 * NP], inv[3 * NP:4 * NP]
        nf = _seg_pass(x1, x2, (h1p, n1, a1p), (h2p, n2, a2p),
                       onesKP, onesKP, iD1, iD2, KP, NP, FTOT)
        out = _seg_pass(nf[:NP], nf[NP:], (n1, h1p, a1p), (n2, h2p, a2p),
                        onesNP, onesNP, iB1, iB2, NP, KP, FTOT)
        o1 = _leaky(out[:K] @ W)
        o2 = _leaky(out[KP:KP + K] @ W)
        return o1, o2

    g1o, g2o = he_layer(x1, x2, h1p, a1p, h2p, a2p, K1, W2)
    e1, m1b, k1b = _pool(g1o, K2, p2)
    e2, m2b, k2b = _pool(g2o, K2, p2)
    h1p2 = m1b[h1p]; a1p2 = a1p * k1b[h1p]
    h2p2 = m2b[h2p]; a2p2 = a2p * k2b[h2p]
    x1, x2 = _cross(e1, e2, Wc2)
    s2 = jnp.concatenate([_readout(x1, Wr2), _readout(x2, Wr2)], axis=1)

    g1o, g2o = he_layer(x1, x2, h1p2, a1p2, h2p2, a2p2, K2, W3)
    e1, _, _ = _pool(g1o, K3, p3)
    e2, _, _ = _pool(g2o, K3, p3)
    x1, x2 = _cross(e1, e2, Wc3)
    s3 = jnp.concatenate([_readout(x1, Wr3), _readout(x2, Wr3)], axis=1)

    scores = jnp.concatenate([s0, s1, s2, s3], axis=1)
    hmid = _leaky(scores @ Wm1 + bm1)
    return hmid @ Wm2 + bm2
